# baseline (speedup 1.0000x reference)
import functools

import jax
import jax.numpy as jnp
from jax import lax
from jax.experimental import pallas as pl
from jax.experimental.pallas import tpu as pltpu
from jax.experimental.pallas import tpu_sc as plsc

_NSUB = 16
_DW = 8
_ZR = 40


def _sc_aggregate(table, dst3, src3, zrows, zrows8, ones_hbm,
                  n_nodes, h, chunks, k):
    nblk = n_nodes // _ZR
    zsteps = (nblk + _NSUB - 1) // _NSUB

    mesh = plsc.VectorSubcoreMesh(core_axis_name="c", subcore_axis_name="s")

    @functools.partial(
        pl.kernel,
        out_type=(
            jax.ShapeDtypeStruct((2 * n_nodes, h), jnp.float32),
            jax.ShapeDtypeStruct((n_nodes, _DW), jnp.float32),
        ),
        mesh=mesh,
        compiler_params=pltpu.CompilerParams(use_tc_tiling_on_sc=False),
        scratch_types=[
            pltpu.VMEM((chunks, k), jnp.int32),
            pltpu.VMEM((chunks, k), jnp.int32),
            pltpu.VMEM((k, h), jnp.float32),
            pltpu.VMEM((k, h), jnp.float32),
            pltpu.VMEM((k, _DW), jnp.float32),
            pltpu.VMEM_SHARED((n_nodes + 8, h), jnp.float32),
            pltpu.VMEM_SHARED((n_nodes + 8, _DW), jnp.float32),
            pltpu.SemaphoreType.DMA,
            pltpu.SemaphoreType.DMA,
            pltpu.SemaphoreType.DMA,
            pltpu.SemaphoreType.DMA,
            pltpu.SemaphoreType.DMA,
        ],
    )
    def body(table_hbm, dst_hbm, src_hbm, zero_hbm, zero8_hbm, ones_in,
             out_hbm, deg_hbm,
             dst_v, src_v, rows0_v, rows1_v, ones_v,
             acc_sh, deg_sh, sg0, ss0, ss1, sd0, sd1):
        c = lax.axis_index("c")
        s = lax.axis_index("s")

        zstage = rows0_v.at[pl.ds(0, _ZR)]
        dstage = ones_v.at[pl.ds(0, _ZR)]
        pltpu.sync_copy(zero_hbm, zstage)
        pltpu.sync_copy(zero8_hbm, dstage)

        def za_desc(z):
            return pltpu.make_async_copy(
                zstage, acc_sh.at[pl.ds((s + z * _NSUB) * _ZR, _ZR)], ss0)

        def zd_desc(z):
            return pltpu.make_async_copy(
                dstage, deg_sh.at[pl.ds((s + z * _NSUB) * _ZR, _ZR)], ss1)

        pltpu.sync_copy(dst_hbm.at[s], dst_v)
        pltpu.sync_copy(src_hbm.at[s], src_v)

        @pl.loop(0, (zsteps + 3) // 4)
        def _(w):
            for i in range(4):
                z = w * 4 + i

                @pl.when(s + z * _NSUB < nblk)
                def _():
                    za_desc(z).start()
                    zd_desc(z).start()

            for i in range(4):
                z = w * 4 + i

                @pl.when(s + z * _NSUB < nblk)
                def _():
                    za_desc(z).wait()
                    zd_desc(z).wait()

        @pl.when(s == 0)
        def _():
            pltpu.sync_copy(rows0_v.at[pl.ds(0, 8)],
                            acc_sh.at[pl.ds(n_nodes, 8)])
            pltpu.sync_copy(ones_v.at[pl.ds(0, 8)],
                            deg_sh.at[pl.ds(n_nodes, 8)])

        pltpu.sync_copy(ones_in, ones_v)

        plsc.subcore_barrier()

        core_table = table_hbm.at[pl.ds(c * n_nodes, n_nodes)]

        def g_desc(j, buf, sem):
            return pltpu.make_async_copy(core_table.at[src_v.at[j]], buf, sem)

        def s_desc(j, buf, sem):
            return pltpu.make_async_copy(buf, acc_sh.at[dst_v.at[j]], sem)

        half = chunks // 2

        def d_desc(j, sem):
            return pltpu.make_async_copy(ones_v, deg_sh.at[dst_v.at[j]], sem)

        npairs = chunks // 2

        g_desc(0, rows0_v, sg0).start()

        @pl.loop(0, npairs)
        def _(g):
            j0 = 2 * g
            g_desc(j0, rows0_v, sg0).wait()
            s_desc(j0, rows0_v, ss0).start(add=True)

            @pl.when(c == 0)
            def _():
                d_desc(j0, sd0).start(add=True)

            @pl.when(g > 0)
            def _():
                s_desc(j0 - 1, rows1_v, ss1).wait()

                @pl.when(c == 0)
                def _():
                    d_desc(j0 - 1, sd1).wait()

            g_desc(j0 + 1, rows1_v, sg0).start()
            g_desc(j0 + 1, rows1_v, sg0).wait()
            s_desc(j0 + 1, rows1_v, ss1).start(add=True)

            @pl.when(c == 0)
            def _():
                d_desc(j0 + 1, sd1).start(add=True)

            s_desc(j0, rows0_v, ss0).wait()

            @pl.when(c == 0)
            def _():
                d_desc(j0, sd0).wait()

            @pl.when(g < npairs - 1)
            def _():
                g_desc(j0 + 2, rows0_v, sg0).start()

        last = 2 * npairs - 1
        s_desc(last, rows1_v, ss1).wait()

        @pl.when(c == 0)
        def _():
            d_desc(last, sd1).wait()

        if chunks % 2 == 1:
            g_desc(chunks - 1, rows0_v, sg0).start()
            g_desc(chunks - 1, rows0_v, sg0).wait()
            s_desc(chunks - 1, rows0_v, ss0).start(add=True)
            s_desc(chunks - 1, rows0_v, ss0).wait()

            @pl.when(c == 0)
            def _():
                d_desc(chunks - 1, sd0).start(add=True)
                d_desc(chunks - 1, sd0).wait()

        plsc.subcore_barrier()

        @pl.loop(0, zsteps)
        def _(z):
            bid = s + z * _NSUB

            @pl.when(bid < nblk)
            def _():
                base = bid * _ZR
                pltpu.sync_copy(acc_sh.at[pl.ds(base, _ZR)], zstage)
                pltpu.sync_copy(zstage,
                                out_hbm.at[pl.ds(c * n_nodes + base, _ZR)])
                @pl.when(c == 0)
                def _():
                    pltpu.sync_copy(deg_sh.at[pl.ds(base, _ZR)], dstage)
                    pltpu.sync_copy(dstage, deg_hbm.at[pl.ds(base, _ZR)])

    return body(table, dst3, src3, zrows, zrows8, ones_hbm)


def _tc_self_matmul(x, w, block_rows):
    n, d = x.shape
    dim = w.shape[1]

    def mm(x_ref, w_ref, o_ref):
        o_ref[...] = jnp.maximum(
            jnp.dot(x_ref[...], w_ref[...], preferred_element_type=jnp.float32),
            0.0)

    return pl.pallas_call(
        mm,
        grid=(n // block_rows,),
        in_specs=[
            pl.BlockSpec((block_rows, d), lambda i: (i, 0)),
            pl.BlockSpec((d, dim), lambda i: (0, 0)),
        ],
        out_specs=pl.BlockSpec((block_rows, dim), lambda i: (i, 0)),
        out_shape=jax.ShapeDtypeStruct((n, dim), jnp.float32),
    )(x, w)


def _tc_neigh_combine(from_self, acc, deg, w_neigh, n_nodes, h, block_rows):
    dim = w_neigh.shape[1]
    half_block = n_nodes // block_rows

    def mm(fs_ref, a0_ref, a1_ref, deg_ref, wn_ref, o_ref):
        acc0 = jnp.dot(a0_ref[...], wn_ref[:h, :],
                       preferred_element_type=jnp.float32)
        acc1 = jnp.dot(a1_ref[...], wn_ref[h:, :],
                       preferred_element_type=jnp.float32)
        recip = 1.0 / jnp.maximum(deg_ref[:, 0], 1e-7)
        o_ref[...] = fs_ref[...] + jnp.maximum((acc0 + acc1) * recip[:, None],
                                               0.0)

    return pl.pallas_call(
        mm,
        grid=(n_nodes // block_rows,),
        in_specs=[
            pl.BlockSpec((block_rows, dim), lambda i: (i, 0)),
            pl.BlockSpec((block_rows, h), lambda i: (i, 0)),
            pl.BlockSpec((block_rows, h), lambda i: (half_block + i, 0)),
            pl.BlockSpec((block_rows, _DW), lambda i: (i, 0)),
            pl.BlockSpec((2 * h, dim), lambda i: (0, 0)),
        ],
        out_specs=pl.BlockSpec((block_rows, dim), lambda i: (i, 0)),
        out_shape=jax.ShapeDtypeStruct((n_nodes, dim), jnp.float32),
    )(from_self, acc, acc, deg, w_neigh)


def kernel(self_embedding, neigh_embedding, edge_index, W_self, W_neigh):
    n_nodes, d_feat = neigh_embedding.shape
    n_edges = edge_index.shape[1]
    h = d_feat // 2
    e_per_sub = n_edges // _NSUB
    k = 80
    chunks = -(-e_per_sub // k)
    pad = chunks * k - e_per_sub
    assert n_edges % _NSUB == 0 and n_nodes % _ZR == 0 and _ZR <= k

    dst = edge_index[0].astype(jnp.int32).reshape(_NSUB, e_per_sub)
    src = edge_index[1].astype(jnp.int32).reshape(_NSUB, e_per_sub)
    if pad:
        dst = jnp.concatenate(
            [dst, jnp.full((_NSUB, pad), n_nodes, jnp.int32)], axis=1)
        src = jnp.concatenate(
            [src, jnp.zeros((_NSUB, pad), jnp.int32)], axis=1)
    dst3 = dst.reshape(_NSUB, chunks, k)
    src3 = src.reshape(_NSUB, chunks, k)

    table = jnp.concatenate([neigh_embedding[:, :h], neigh_embedding[:, h:]],
                            axis=0)
    zrows = jnp.zeros((_ZR, h), jnp.float32)
    zrows8 = jnp.zeros((_ZR, _DW), jnp.float32)
    ones_hbm = jnp.ones((k, _DW), jnp.float32)

    summed, deg = _sc_aggregate(table, dst3, src3, zrows, zrows8, ones_hbm,
                                n_nodes, h, chunks, k)
    from_self = _tc_self_matmul(self_embedding, W_self, block_rows=1000)
    return _tc_neigh_combine(from_self, summed, deg, W_neigh, n_nodes, h,
                             block_rows=1000)


# speedup vs baseline: 1.0036x; 1.0036x over previous
import functools

import jax
import jax.numpy as jnp
from jax import lax
from jax.experimental import pallas as pl
from jax.experimental.pallas import tpu as pltpu
from jax.experimental.pallas import tpu_sc as plsc

_NSUB = 16
_DW = 8
_ZR = 40


def _sc_aggregate(table, dst3, src3, zrows, zrows8, ones_hbm,
                  n_nodes, h, chunks, k):
    nblk = n_nodes // _ZR
    zsteps = (nblk + _NSUB - 1) // _NSUB

    mesh = plsc.VectorSubcoreMesh(core_axis_name="c", subcore_axis_name="s")

    @functools.partial(
        pl.kernel,
        out_type=(
            jax.ShapeDtypeStruct((2 * n_nodes, h), jnp.float32),
            jax.ShapeDtypeStruct((n_nodes, _DW), jnp.float32),
        ),
        mesh=mesh,
        compiler_params=pltpu.CompilerParams(use_tc_tiling_on_sc=False),
        scratch_types=[
            pltpu.VMEM((chunks, k), jnp.int32),
            pltpu.VMEM((chunks, k), jnp.int32),
            pltpu.VMEM((k, h), jnp.float32),
            pltpu.VMEM((k, h), jnp.float32),
            pltpu.VMEM((k, _DW), jnp.float32),
            pltpu.VMEM_SHARED((n_nodes + 8, h), jnp.float32),
            pltpu.VMEM_SHARED((n_nodes + 8, _DW), jnp.float32),
            pltpu.SemaphoreType.DMA,
            pltpu.SemaphoreType.DMA,
            pltpu.SemaphoreType.DMA,
            pltpu.SemaphoreType.DMA,
            pltpu.SemaphoreType.DMA,
        ],
    )
    def body(table_hbm, dst_hbm, src_hbm, zero_hbm, zero8_hbm, ones_in,
             out_hbm, deg_hbm,
             dst_v, src_v, rows0_v, rows1_v, ones_v,
             acc_sh, deg_sh, sg0, ss0, ss1, sd0, sd1):
        c = lax.axis_index("c")
        s = lax.axis_index("s")

        pltpu.sync_copy(dst_hbm.at[s], dst_v)
        pltpu.sync_copy(src_hbm.at[s], src_v)

        zstage = rows0_v.at[pl.ds(0, _ZR)]
        dstage = ones_v.at[pl.ds(0, _ZR)]
        pltpu.sync_copy(zero_hbm, zstage)
        pltpu.sync_copy(zero8_hbm, dstage)

        @pl.loop(0, zsteps)
        def _(z):
            bid = s + z * _NSUB

            @pl.when(bid < nblk)
            def _():
                pltpu.sync_copy(zstage, acc_sh.at[pl.ds(bid * _ZR, _ZR)])
                pltpu.sync_copy(dstage, deg_sh.at[pl.ds(bid * _ZR, _ZR)])

        @pl.when(s == 0)
        def _():
            pltpu.sync_copy(rows0_v.at[pl.ds(0, 8)],
                            acc_sh.at[pl.ds(n_nodes, 8)])
            pltpu.sync_copy(ones_v.at[pl.ds(0, 8)],
                            deg_sh.at[pl.ds(n_nodes, 8)])

        pltpu.sync_copy(ones_in, ones_v)

        plsc.subcore_barrier()

        core_table = table_hbm.at[pl.ds(c * n_nodes, n_nodes)]

        def g_desc(j, buf, sem):
            return pltpu.make_async_copy(core_table.at[src_v.at[j]], buf, sem)

        def s_desc(j, buf, sem):
            return pltpu.make_async_copy(buf, acc_sh.at[dst_v.at[j]], sem)

        half = chunks // 2

        def d_desc(j, sem):
            return pltpu.make_async_copy(ones_v, deg_sh.at[dst_v.at[j]], sem)

        npairs = chunks // 2

        g_desc(0, rows0_v, sg0).start()

        @pl.loop(0, npairs)
        def _(g):
            j0 = 2 * g
            g_desc(j0, rows0_v, sg0).wait()
            s_desc(j0, rows0_v, ss0).start(add=True)

            @pl.when(c == 0)
            def _():
                d_desc(j0, sd0).start(add=True)

            @pl.when(g > 0)
            def _():
                s_desc(j0 - 1, rows1_v, ss1).wait()

                @pl.when(c == 0)
                def _():
                    d_desc(j0 - 1, sd1).wait()

            g_desc(j0 + 1, rows1_v, sg0).start()
            g_desc(j0 + 1, rows1_v, sg0).wait()
            s_desc(j0 + 1, rows1_v, ss1).start(add=True)

            @pl.when(c == 0)
            def _():
                d_desc(j0 + 1, sd1).start(add=True)

            s_desc(j0, rows0_v, ss0).wait()

            @pl.when(c == 0)
            def _():
                d_desc(j0, sd0).wait()

            @pl.when(g < npairs - 1)
            def _():
                g_desc(j0 + 2, rows0_v, sg0).start()

        last = 2 * npairs - 1
        s_desc(last, rows1_v, ss1).wait()

        @pl.when(c == 0)
        def _():
            d_desc(last, sd1).wait()

        if chunks % 2 == 1:
            g_desc(chunks - 1, rows0_v, sg0).start()
            g_desc(chunks - 1, rows0_v, sg0).wait()
            s_desc(chunks - 1, rows0_v, ss0).start(add=True)
            s_desc(chunks - 1, rows0_v, ss0).wait()

            @pl.when(c == 0)
            def _():
                d_desc(chunks - 1, sd0).start(add=True)
                d_desc(chunks - 1, sd0).wait()

        plsc.subcore_barrier()

        @pl.loop(0, zsteps)
        def _(z):
            bid = s + z * _NSUB

            @pl.when(bid < nblk)
            def _():
                base = bid * _ZR
                pltpu.sync_copy(acc_sh.at[pl.ds(base, _ZR)], zstage)
                pltpu.sync_copy(zstage,
                                out_hbm.at[pl.ds(c * n_nodes + base, _ZR)])
                @pl.when(c == 0)
                def _():
                    pltpu.sync_copy(deg_sh.at[pl.ds(base, _ZR)], dstage)
                    pltpu.sync_copy(dstage, deg_hbm.at[pl.ds(base, _ZR)])

    return body(table, dst3, src3, zrows, zrows8, ones_hbm)


def _tc_self_matmul(x, w, block_rows):
    n, d = x.shape
    dim = w.shape[1]

    def mm(x_ref, w_ref, o_ref):
        o_ref[...] = jnp.maximum(
            jnp.dot(x_ref[...], w_ref[...], preferred_element_type=jnp.float32),
            0.0)

    return pl.pallas_call(
        mm,
        grid=(n // block_rows,),
        in_specs=[
            pl.BlockSpec((block_rows, d), lambda i: (i, 0)),
            pl.BlockSpec((d, dim), lambda i: (0, 0)),
        ],
        out_specs=pl.BlockSpec((block_rows, dim), lambda i: (i, 0)),
        out_shape=jax.ShapeDtypeStruct((n, dim), jnp.float32),
    )(x, w)


def _tc_neigh_combine(from_self, w_self, acc, deg, w_neigh, n_nodes, h,
                      block_rows):
    dim = w_neigh.shape[1]
    half_block = n_nodes // block_rows

    def mm(fs_ref, ws_ref, a0_ref, a1_ref, deg_ref, wn_ref, o_ref):
        fs = jnp.maximum(
            jnp.dot(fs_ref[...], ws_ref[...],
                    preferred_element_type=jnp.float32), 0.0)
        acc0 = jnp.dot(a0_ref[...], wn_ref[:h, :],
                       preferred_element_type=jnp.float32)
        acc1 = jnp.dot(a1_ref[...], wn_ref[h:, :],
                       preferred_element_type=jnp.float32)
        recip = 1.0 / jnp.maximum(deg_ref[:, 0], 1e-7)
        o_ref[...] = fs + jnp.maximum((acc0 + acc1) * recip[:, None], 0.0)

    return pl.pallas_call(
        mm,
        grid=(n_nodes // block_rows,),
        in_specs=[
            pl.BlockSpec((block_rows, dim), lambda i: (i, 0)),
            pl.BlockSpec((dim, dim), lambda i: (0, 0)),
            pl.BlockSpec((block_rows, h), lambda i: (i, 0)),
            pl.BlockSpec((block_rows, h), lambda i: (half_block + i, 0)),
            pl.BlockSpec((block_rows, _DW), lambda i: (i, 0)),
            pl.BlockSpec((2 * h, dim), lambda i: (0, 0)),
        ],
        out_specs=pl.BlockSpec((block_rows, dim), lambda i: (i, 0)),
        out_shape=jax.ShapeDtypeStruct((n_nodes, dim), jnp.float32),
    )(from_self, w_self, acc, acc, deg, w_neigh)


def kernel(self_embedding, neigh_embedding, edge_index, W_self, W_neigh):
    n_nodes, d_feat = neigh_embedding.shape
    n_edges = edge_index.shape[1]
    h = d_feat // 2
    e_per_sub = n_edges // _NSUB
    k = 80
    chunks = -(-e_per_sub // k)
    pad = chunks * k - e_per_sub
    assert n_edges % _NSUB == 0 and n_nodes % _ZR == 0 and _ZR <= k

    dst = edge_index[0].astype(jnp.int32).reshape(_NSUB, e_per_sub)
    src = edge_index[1].astype(jnp.int32).reshape(_NSUB, e_per_sub)
    if pad:
        dst = jnp.concatenate(
            [dst, jnp.full((_NSUB, pad), n_nodes, jnp.int32)], axis=1)
        src = jnp.concatenate(
            [src, jnp.zeros((_NSUB, pad), jnp.int32)], axis=1)
    dst3 = dst.reshape(_NSUB, chunks, k)
    src3 = src.reshape(_NSUB, chunks, k)

    table = jnp.concatenate([neigh_embedding[:, :h], neigh_embedding[:, h:]],
                            axis=0)
    zrows = jnp.zeros((_ZR, h), jnp.float32)
    zrows8 = jnp.zeros((_ZR, _DW), jnp.float32)
    ones_hbm = jnp.ones((k, _DW), jnp.float32)

    summed, deg = _sc_aggregate(table, dst3, src3, zrows, zrows8, ones_hbm,
                                n_nodes, h, chunks, k)
    return _tc_neigh_combine(self_embedding, W_self, summed, deg, W_neigh,
                             n_nodes, h, block_rows=1000)
